# baseline (speedup 1.0000x reference)
import jax
import jax.numpy as jnp
from jax.experimental import pallas as pl


def _add_body(x_ref, e_ref, o_ref):
    o_ref[...] = x_ref[...] + e_ref[...]


def kernel(inputs, embeddings):
    batch, seq_len, dim = inputs.shape
    blk = 512
    nseq = seq_len // blk
    flat = inputs.reshape(batch * seq_len, dim)
    out = pl.pallas_call(
        _add_body,
        grid=(nseq * batch,),
        in_specs=[
            pl.BlockSpec((blk, dim), lambda j: ((j % batch) * nseq + j // batch, 0)),
            pl.BlockSpec((blk, dim), lambda j: (j // batch, 0)),
        ],
        out_specs=pl.BlockSpec((blk, dim), lambda j: ((j % batch) * nseq + j // batch, 0)),
        out_shape=jax.ShapeDtypeStruct((batch * seq_len, dim), inputs.dtype),
    )(flat, embeddings[:seq_len])
    return out.reshape(batch, seq_len, dim)
